# baseline (speedup 1.0000x reference)
import functools

import jax
import jax.numpy as jnp
from jax import lax
from jax.experimental import pallas as pl
from jax.experimental.pallas import tpu as pltpu
from jax.experimental.pallas import tpu_sc as plsc

BATCH = 16384
SEQ = 50
DIM = 65
PAD_DIM = 128
NUM_ROWS = BATCH * SEQ
NUM_WORKERS = 32
PER_WORKER = NUM_ROWS // NUM_WORKERS
CHUNK = 128
NUM_CHUNKS = PER_WORKER // CHUNK

PAD_BLK = 2000
FMT_B = 8


def _pad_kernel(w_ref, o_ref):
    o_ref[:, :DIM] = w_ref[...]


def _tc_pad(weight):
    return pl.pallas_call(
        _pad_kernel,
        grid=(weight.shape[0] // PAD_BLK,),
        in_specs=[pl.BlockSpec((PAD_BLK, DIM), lambda i: (i, 0))],
        out_specs=pl.BlockSpec((PAD_BLK, PAD_DIM), lambda i: (i, 0)),
        out_shape=jax.ShapeDtypeStruct((weight.shape[0], PAD_DIM), jnp.float32),
    )(weight)


def _gather_kernel(idx_hbm, table_hbm, out_hbm, idx_v, rows_v, sem):
    wid = lax.axis_index("s") * 2 + lax.axis_index("c")
    base = wid * PER_WORKER
    pltpu.sync_copy(idx_hbm.at[pl.ds(wid * NUM_CHUNKS, NUM_CHUNKS)], idx_v)

    def body(j, _):
        pltpu.async_copy(table_hbm.at[idx_v.at[j]], rows_v, sem).wait()
        pltpu.sync_copy(rows_v, out_hbm.at[pl.ds(base + j * CHUNK, CHUNK)])
        return 0

    lax.fori_loop(0, NUM_CHUNKS, body, 0)


def _sc_gather(idx, table):
    mesh = plsc.VectorSubcoreMesh(core_axis_name="c", subcore_axis_name="s")
    k = functools.partial(
        pl.kernel,
        mesh=mesh,
        out_type=jax.ShapeDtypeStruct((NUM_ROWS, PAD_DIM), jnp.float32),
        scratch_types=[
            pltpu.VMEM((NUM_CHUNKS, CHUNK), jnp.int32),
            pltpu.VMEM((CHUNK, PAD_DIM), jnp.float32),
            pltpu.SemaphoreType.DMA,
        ],
    )(_gather_kernel)
    return k(idx, table)


def _fmt_kernel(slab_ref, o_ref):
    for k in range(FMT_B):
        o_ref[k] = slab_ref[pl.ds(SEQ * k, SEQ), :DIM]


def _tc_format(slab):
    return pl.pallas_call(
        _fmt_kernel,
        grid=(BATCH // FMT_B,),
        in_specs=[pl.BlockSpec((FMT_B * SEQ, PAD_DIM), lambda i: (i, 0))],
        out_specs=pl.BlockSpec((FMT_B, SEQ, DIM), lambda i: (i, 0, 0)),
        out_shape=jax.ShapeDtypeStruct((BATCH, SEQ, DIM), jnp.float32),
    )(slab)


def kernel(indices, weight):
    table = _tc_pad(weight.astype(jnp.float32))
    idx = indices.reshape(NUM_ROWS // CHUNK, CHUNK).astype(jnp.int32)
    slab = _sc_gather(idx, table)
    return _tc_format(slab)


# speedup vs baseline: 1.4505x; 1.4505x over previous
import functools

import jax
import jax.numpy as jnp
from jax import lax
from jax.experimental import pallas as pl
from jax.experimental.pallas import tpu as pltpu
from jax.experimental.pallas import tpu_sc as plsc

BATCH = 16384
SEQ = 50
DIM = 65
PAD_DIM = 128
NUM_ROWS = BATCH * SEQ
NUM_WORKERS = 32
PER_WORKER = NUM_ROWS // NUM_WORKERS
CHUNK = 128
NUM_CHUNKS = PER_WORKER // CHUNK

PAD_BLK = 2000
FMT_B = 8


def _pad_kernel(w_ref, o_ref):
    o_ref[:, :DIM] = w_ref[...]


def _tc_pad(weight):
    return pl.pallas_call(
        _pad_kernel,
        grid=(weight.shape[0] // PAD_BLK,),
        in_specs=[pl.BlockSpec((PAD_BLK, DIM), lambda i: (i, 0))],
        out_specs=pl.BlockSpec((PAD_BLK, PAD_DIM), lambda i: (i, 0)),
        out_shape=jax.ShapeDtypeStruct((weight.shape[0], PAD_DIM), jnp.float32),
    )(weight)


def _gather_kernel(idx_hbm, table_hbm, out_hbm, idx_v, rows_v, sem):
    wid = lax.axis_index("s") * 2 + lax.axis_index("c")
    base = wid * PER_WORKER
    pltpu.sync_copy(idx_hbm.at[pl.ds(wid * NUM_CHUNKS, NUM_CHUNKS)], idx_v)

    def body(j, _):
        pltpu.async_copy(table_hbm.at[idx_v.at[j]], rows_v, sem).wait()
        pltpu.sync_copy(rows_v, out_hbm.at[pl.ds(base + j * CHUNK, CHUNK)])
        return 0

    lax.fori_loop(0, NUM_CHUNKS, body, 0)


def _sc_gather(idx, table):
    mesh = plsc.VectorSubcoreMesh(core_axis_name="c", subcore_axis_name="s")
    k = functools.partial(
        pl.kernel,
        mesh=mesh,
        out_type=jax.ShapeDtypeStruct((NUM_ROWS, PAD_DIM), jnp.float32),
        scratch_types=[
            pltpu.VMEM((NUM_CHUNKS, CHUNK), jnp.int32),
            pltpu.VMEM((CHUNK, PAD_DIM), jnp.float32),
            pltpu.SemaphoreType.DMA,
        ],
    )(_gather_kernel)
    return k(idx, table)


def _fmt_kernel(slab_ref, o_ref):
    for k in range(FMT_B):
        o_ref[k] = slab_ref[pl.ds(SEQ * k, SEQ), :DIM]


def _tc_format(slab):
    return pl.pallas_call(
        _fmt_kernel,
        grid=(BATCH // FMT_B,),
        in_specs=[pl.BlockSpec((FMT_B * SEQ, PAD_DIM), lambda i: (i, 0))],
        out_specs=pl.BlockSpec((FMT_B, SEQ, DIM), lambda i: (i, 0, 0)),
        out_shape=jax.ShapeDtypeStruct((BATCH, SEQ, DIM), jnp.float32),
    )(slab)


def kernel(indices, weight):
    table = _tc_pad(weight.astype(jnp.float32))
    idx = indices.reshape(NUM_ROWS // CHUNK, CHUNK).astype(jnp.int32)
    slab = _sc_gather(idx, table)
    return slab[:, :DIM].reshape(BATCH, SEQ, DIM)
